# baseline (speedup 1.0000x reference)
import functools

import jax
import jax.numpy as jnp
from jax import lax
from jax.experimental import pallas as pl
from jax.experimental.pallas import tpu as pltpu
from jax.experimental.pallas import tpu_sc as plsc

_B = 16384
_D = 128
_GAMMA = 12.0
_NC = 2
_NS = 16
_NW = _NC * _NS
_BPW = _B // _NW
_CH = 128
_CHUNKS = (64, 128, 128, 128, 64)


def _tec_body(hid_hbm, rid_hbm, tid_hbm, ent_hbm, rel_hbm, out_hbm,
              ih_v, ir_v, it_v, h_v, r_v, t_v, part_v, out_v, sem0, sem1):
    wid = lax.axis_index("s") * _NC + lax.axis_index("c")
    base = wid * _BPW

    lane = lax.broadcasted_iota(jnp.int32, (16,), 0)

    cps = (
        pltpu.async_copy(hid_hbm.at[pl.ds(base, _BPW)], ih_v, sem0),
        pltpu.async_copy(rid_hbm.at[pl.ds(base, _BPW)], ir_v, sem0),
        pltpu.async_copy(tid_hbm.at[pl.ds(base, _BPW)], it_v, sem0),
    )
    for cp in cps:
        cp.wait()

    offs = []
    o = 0
    for n in _CHUNKS:
        offs.append(o)
        o += n
    sems = (sem0, sem1)

    def issue(ck):
        b = ck % 2
        n = _CHUNKS[ck]
        cbase = offs[ck]
        sem = sems[b]
        return (
            pltpu.async_copy(ent_hbm.at[ih_v.at[pl.ds(cbase, n)]],
                             h_v.at[b, pl.ds(0, n)], sem),
            pltpu.async_copy(rel_hbm.at[ir_v.at[pl.ds(cbase, n)]],
                             r_v.at[b, pl.ds(0, n)], sem),
            pltpu.async_copy(ent_hbm.at[it_v.at[pl.ds(cbase, n)]],
                             t_v.at[b, pl.ds(0, n)], sem),
        )

    pending = issue(0)
    for ck, n in enumerate(_CHUNKS):
        b = ck % 2
        nxt = issue(ck + 1) if ck + 1 < len(_CHUNKS) else None
        for cp in pending:
            cp.wait()
        pending = nxt

        def row_fn(rr, carry):
            acc = None
            for c in range(_D // 16):
                hv = h_v[b, rr, pl.ds(c * 16, 16)]
                rv = r_v[b, rr, pl.ds(c * 16, 16)]
                tv = t_v[b, rr, pl.ds(c * 16, 16)]
                term = jnp.abs(hv + rv - tv)
                acc = term if acc is None else acc + term
            part_v[pl.ds(rr * 16, 16)] = acc
            return carry

        lax.fori_loop(0, n, row_fn, 0, unroll=2)

        def grp_fn(g, carry):
            vec = jnp.zeros((16,), jnp.float32)
            for j in range(16):
                pj = part_v[pl.ds(g * 256 + j * 16, 16)]
                sj = jnp.sum(pj)
                vec = jnp.where(lane == j, _GAMMA - sj, vec)
            out_v[pl.ds(offs[ck] + g * 16, 16)] = vec
            return carry

        lax.fori_loop(0, n // 16, grp_fn, 0, unroll=False)

    pltpu.sync_copy(out_v, out_hbm.at[pl.ds(base, _BPW)])


@functools.partial(
    pl.kernel,
    out_type=jax.ShapeDtypeStruct((_B,), jnp.float32),
    mesh=plsc.VectorSubcoreMesh(core_axis_name="c", subcore_axis_name="s"),
    compiler_params=pltpu.CompilerParams(needs_layout_passes=False),
    scratch_types=[
        pltpu.VMEM((_BPW,), jnp.int32),
        pltpu.VMEM((_BPW,), jnp.int32),
        pltpu.VMEM((_BPW,), jnp.int32),
        pltpu.VMEM((2, _CH, _D), jnp.float32),
        pltpu.VMEM((2, _CH, _D), jnp.float32),
        pltpu.VMEM((2, _CH, _D), jnp.float32),
        pltpu.VMEM((_CH * 16,), jnp.float32),
        pltpu.VMEM((_BPW,), jnp.float32),
        pltpu.SemaphoreType.DMA,
        pltpu.SemaphoreType.DMA,
    ],
)
def _score_sc(hid_hbm, rid_hbm, tid_hbm, ent_hbm, rel_hbm, out_hbm, *scratch):
    _tec_body(hid_hbm, rid_hbm, tid_hbm, ent_hbm, rel_hbm, out_hbm, *scratch)


@jax.jit
def _run(sample, entity_embedding, relation_embedding):
    samT = jnp.transpose(sample)
    out = _score_sc(samT[0], samT[1], samT[2],
                    entity_embedding, relation_embedding)
    return out.reshape(_B, 1)


def kernel(idx, sample, entity_embedding, relation_embedding):
    return _run(sample, entity_embedding, relation_embedding)


# speedup vs baseline: 1.0213x; 1.0213x over previous
import functools

import jax
import jax.numpy as jnp
from jax import lax
from jax.experimental import pallas as pl
from jax.experimental.pallas import tpu as pltpu
from jax.experimental.pallas import tpu_sc as plsc

_B = 16384
_D = 128
_GAMMA = 12.0
_NC = 2
_NS = 16
_NW = _NC * _NS
_BPW = _B // _NW
_CH = 128
_CHUNKS = (64, 128, 128, 128, 64)


def _tec_body(hid_hbm, rid_hbm, tid_hbm, ent_hbm, rel_hbm, out_hbm,
              ih_v, ir_v, it_v, h_v, r_v, t_v, part_v, out_v, sem0, sem1):
    wid = lax.axis_index("s") * _NC + lax.axis_index("c")
    base = wid * _BPW

    lane = lax.broadcasted_iota(jnp.int32, (16,), 0)

    cps = (
        pltpu.async_copy(hid_hbm.at[pl.ds(base, _BPW)], ih_v, sem0),
        pltpu.async_copy(rid_hbm.at[pl.ds(base, _BPW)], ir_v, sem0),
        pltpu.async_copy(tid_hbm.at[pl.ds(base, _BPW)], it_v, sem0),
    )
    for cp in cps:
        cp.wait()

    offs = []
    o = 0
    for n in _CHUNKS:
        offs.append(o)
        o += n
    sems = (sem0, sem1)

    def issue(ck):
        b = ck % 2
        n = _CHUNKS[ck]
        cbase = offs[ck]
        sem = sems[b]
        return (
            pltpu.async_copy(ent_hbm.at[ih_v.at[pl.ds(cbase, n)]],
                             h_v.at[b, pl.ds(0, n)], sem),
            pltpu.async_copy(rel_hbm.at[ir_v.at[pl.ds(cbase, n)]],
                             r_v.at[b, pl.ds(0, n)], sem),
            pltpu.async_copy(ent_hbm.at[it_v.at[pl.ds(cbase, n)]],
                             t_v.at[b, pl.ds(0, n)], sem),
        )

    pending = issue(0)
    for ck, n in enumerate(_CHUNKS):
        b = ck % 2
        nxt = issue(ck + 1) if ck + 1 < len(_CHUNKS) else None
        for cp in pending:
            cp.wait()
        pending = nxt

        def row_fn(rr, carry):
            acc = None
            for c in range(_D // 16):
                hv = h_v[b, rr, pl.ds(c * 16, 16)]
                rv = r_v[b, rr, pl.ds(c * 16, 16)]
                tv = t_v[b, rr, pl.ds(c * 16, 16)]
                term = jnp.abs(hv + rv - tv)
                acc = term if acc is None else acc + term
            part_v[pl.ds(rr * 16, 16)] = acc
            return carry

        lax.fori_loop(0, n, row_fn, 0, unroll=False)

        def grp_fn(g, carry):
            vec = jnp.zeros((16,), jnp.float32)
            for j in range(16):
                pj = part_v[pl.ds(g * 256 + j * 16, 16)]
                sj = jnp.sum(pj)
                vec = jnp.where(lane == j, _GAMMA - sj, vec)
            out_v[pl.ds(offs[ck] + g * 16, 16)] = vec
            return carry

        lax.fori_loop(0, n // 16, grp_fn, 0, unroll=False)

    pltpu.sync_copy(out_v, out_hbm.at[pl.ds(base, _BPW)])


@functools.partial(
    pl.kernel,
    out_type=jax.ShapeDtypeStruct((_B,), jnp.float32),
    mesh=plsc.VectorSubcoreMesh(core_axis_name="c", subcore_axis_name="s"),
    compiler_params=pltpu.CompilerParams(needs_layout_passes=False),
    scratch_types=[
        pltpu.VMEM((_BPW,), jnp.int32),
        pltpu.VMEM((_BPW,), jnp.int32),
        pltpu.VMEM((_BPW,), jnp.int32),
        pltpu.VMEM((2, _CH, _D), jnp.float32),
        pltpu.VMEM((2, _CH, _D), jnp.float32),
        pltpu.VMEM((2, _CH, _D), jnp.float32),
        pltpu.VMEM((_CH * 16,), jnp.float32),
        pltpu.VMEM((_BPW,), jnp.float32),
        pltpu.SemaphoreType.DMA,
        pltpu.SemaphoreType.DMA,
    ],
)
def _score_sc(hid_hbm, rid_hbm, tid_hbm, ent_hbm, rel_hbm, out_hbm, *scratch):
    _tec_body(hid_hbm, rid_hbm, tid_hbm, ent_hbm, rel_hbm, out_hbm, *scratch)


@jax.jit
def _run(sample, entity_embedding, relation_embedding):
    out = _score_sc(sample[:, 0], sample[:, 1], sample[:, 2],
                    entity_embedding, relation_embedding)
    return out.reshape(_B, 1)


def kernel(idx, sample, entity_embedding, relation_embedding):
    return _run(sample, entity_embedding, relation_embedding)


# speedup vs baseline: 1.0321x; 1.0106x over previous
import functools

import jax
import jax.numpy as jnp
from jax import lax
from jax.experimental import pallas as pl
from jax.experimental.pallas import tpu as pltpu
from jax.experimental.pallas import tpu_sc as plsc

_B = 16384
_D = 128
_GAMMA = 12.0
_NC = 2
_NS = 16
_NW = _NC * _NS
_BPW = _B // _NW
_CH = 128
_CHUNKS = (32, 128, 128, 128, 64, 32)


def _tec_body(hid_hbm, rid_hbm, tid_hbm, ent_hbm, rel_hbm, out_hbm,
              ih_v, ir_v, it_v, h_v, r_v, t_v, part_v, out_v, sem0, sem1):
    wid = lax.axis_index("s") * _NC + lax.axis_index("c")
    base = wid * _BPW

    lane = lax.broadcasted_iota(jnp.int32, (16,), 0)

    cps = (
        pltpu.async_copy(hid_hbm.at[pl.ds(base, _BPW)], ih_v, sem0),
        pltpu.async_copy(rid_hbm.at[pl.ds(base, _BPW)], ir_v, sem0),
        pltpu.async_copy(tid_hbm.at[pl.ds(base, _BPW)], it_v, sem0),
    )
    for cp in cps:
        cp.wait()

    offs = []
    o = 0
    for n in _CHUNKS:
        offs.append(o)
        o += n
    sems = (sem0, sem1)

    def issue(ck):
        b = ck % 2
        n = _CHUNKS[ck]
        cbase = offs[ck]
        sem = sems[b]
        return (
            pltpu.async_copy(ent_hbm.at[ih_v.at[pl.ds(cbase, n)]],
                             h_v.at[b, pl.ds(0, n)], sem),
            pltpu.async_copy(rel_hbm.at[ir_v.at[pl.ds(cbase, n)]],
                             r_v.at[b, pl.ds(0, n)], sem),
            pltpu.async_copy(ent_hbm.at[it_v.at[pl.ds(cbase, n)]],
                             t_v.at[b, pl.ds(0, n)], sem),
        )

    pending = issue(0)
    for ck, n in enumerate(_CHUNKS):
        b = ck % 2
        nxt = issue(ck + 1) if ck + 1 < len(_CHUNKS) else None
        for cp in pending:
            cp.wait()
        pending = nxt

        def row_fn(rr, carry):
            acc = None
            for c in range(_D // 16):
                hv = h_v[b, rr, pl.ds(c * 16, 16)]
                rv = r_v[b, rr, pl.ds(c * 16, 16)]
                tv = t_v[b, rr, pl.ds(c * 16, 16)]
                term = jnp.abs(hv + rv - tv)
                acc = term if acc is None else acc + term
            part_v[pl.ds(rr * 16, 16)] = acc
            return carry

        lax.fori_loop(0, n, row_fn, 0, unroll=False)

        def grp_fn(g, carry):
            vec = jnp.zeros((16,), jnp.float32)
            for j in range(16):
                pj = part_v[pl.ds(g * 256 + j * 16, 16)]
                sj = jnp.sum(pj)
                vec = jnp.where(lane == j, _GAMMA - sj, vec)
            out_v[pl.ds(offs[ck] + g * 16, 16)] = vec
            return carry

        lax.fori_loop(0, n // 16, grp_fn, 0, unroll=False)

    pltpu.sync_copy(out_v, out_hbm.at[pl.ds(base, _BPW)])


@functools.partial(
    pl.kernel,
    out_type=jax.ShapeDtypeStruct((_B,), jnp.float32),
    mesh=plsc.VectorSubcoreMesh(core_axis_name="c", subcore_axis_name="s"),
    compiler_params=pltpu.CompilerParams(needs_layout_passes=False),
    scratch_types=[
        pltpu.VMEM((_BPW,), jnp.int32),
        pltpu.VMEM((_BPW,), jnp.int32),
        pltpu.VMEM((_BPW,), jnp.int32),
        pltpu.VMEM((2, _CH, _D), jnp.float32),
        pltpu.VMEM((2, _CH, _D), jnp.float32),
        pltpu.VMEM((2, _CH, _D), jnp.float32),
        pltpu.VMEM((_CH * 16,), jnp.float32),
        pltpu.VMEM((_BPW,), jnp.float32),
        pltpu.SemaphoreType.DMA,
        pltpu.SemaphoreType.DMA,
    ],
)
def _score_sc(hid_hbm, rid_hbm, tid_hbm, ent_hbm, rel_hbm, out_hbm, *scratch):
    _tec_body(hid_hbm, rid_hbm, tid_hbm, ent_hbm, rel_hbm, out_hbm, *scratch)


@jax.jit
def _run(sample, entity_embedding, relation_embedding):
    out = _score_sc(sample[:, 0], sample[:, 1], sample[:, 2],
                    entity_embedding, relation_embedding)
    return out.reshape(_B, 1)


def kernel(idx, sample, entity_embedding, relation_embedding):
    return _run(sample, entity_embedding, relation_embedding)
